# baseline (speedup 1.0000x reference)
import functools

import jax
import jax.numpy as jnp
from jax import lax
from jax.experimental import pallas as pl
from jax.experimental.pallas import tpu as pltpu
from jax.experimental.pallas import tpu_sc as plsc

NU = 30000
NI = 70000
D = 64
NNZ = 1000000
NB = 4096
NLAYERS = 3.0
DECAY = 1e-4

NC = 2
NS = 16
CH = 800
EPAD = 1024000
NPAD = EPAD - NNZ
NDUMU = 704
NDUMI = 1600
AU = NU + 720
AI = NI + 2000
SA = AU // NS
EW = EPAD // NS
NCH = EW // CH
GA = 4
GB = 10
TA = NCH // GA
TB = NCH // GB

_mesh = plsc.VectorSubcoreMesh(core_axis_name="c", subcore_axis_name="s",
                               num_cores=NC, num_subcores=NS)
_sc_params = pltpu.CompilerParams(use_tc_tiling_on_sc=False)


def _sc_deg_body(rows_s_ref, deg_ref, didx_a, didx_b, onesv, zdeg, dacc, semd):
    c = lax.axis_index("c")
    s = lax.axis_index("s")
    didx = (didx_a, didx_b)

    def _init(t, _):
        onesv[pl.ds(t * 16, 16)] = jnp.full((16,), 1.0, jnp.float32)
        return 0
    lax.fori_loop(0, CH // 16, _init, 0)

    def _zdeg(t, _):
        zdeg[pl.ds(t * 16, 16)] = jnp.zeros((16,), jnp.float32)
        return 0
    lax.fori_loop(0, SA // 16, _zdeg, 0)

    pltpu.sync_copy(zdeg, dacc.at[pl.ds(s * SA, SA)])
    plsc.subcore_barrier()

    row0 = (c * NS + s) * (EPAD // (NC * NS) // CH)
    nrows = EPAD // (NC * NS) // CH
    sd = [None, None]
    for q in range(nrows):
        p = q & 1
        if sd[p] is not None:
            sd[p].wait()
        pltpu.sync_copy(rows_s_ref.at[pl.ds(row0 + q, 1)], didx[p])
        sd[p] = pltpu.async_copy(onesv, dacc.at[didx[p].at[0]], semd,
                                 add=True)
    for b in range(2):
        if sd[b] is not None:
            sd[b].wait()
    plsc.subcore_barrier()
    pltpu.sync_copy(dacc.at[pl.ds(s * SA, SA)], zdeg)
    pltpu.sync_copy(zdeg, deg_ref.at[pl.ds(c * AU + s * SA, SA)])


_sc_deg = functools.partial(
    pl.kernel,
    out_type=jax.ShapeDtypeStruct((NC * AU,), jnp.float32),
    mesh=_mesh,
    scratch_types=[
        pltpu.VMEM((1, CH), jnp.int32),
        pltpu.VMEM((1, CH), jnp.int32),
        pltpu.VMEM((CH,), jnp.float32),
        pltpu.VMEM((SA,), jnp.float32),
        pltpu.VMEM_SHARED((AU,), jnp.float32),
        pltpu.SemaphoreType.DMA,
    ],
    compiler_params=_sc_params,
)(_sc_deg_body)


def _sc_spmm_u_body(rows_s_ref, cols_g_ref, item_t_ref,
                    hu_ref,
                    cidx_blk, ridx_blk, gidx_a, gidx_b,
                    rowsv_a, rowsv_b,
                    acc, semga, semgb, semsa, semsb):
    c = lax.axis_index("c")
    s = lax.axis_index("s")
    gidx = (gidx_a, gidx_b)
    rowsv = (rowsv_a, rowsv_b)
    gsem = (semga, semgb)
    ssem = (semsa, semsb)

    def _zrow(r, _):
        rowsv_a[r, pl.ds(0, 16)] = jnp.zeros((16,), jnp.float32)
        rowsv_a[r, pl.ds(16, 16)] = jnp.zeros((16,), jnp.float32)
        return 0
    lax.fori_loop(0, CH, _zrow, 0)

    pltpu.sync_copy(rowsv_a, acc.at[pl.ds(s * SA, CH)])
    pltpu.sync_copy(rowsv_a, acc.at[pl.ds(s * SA + CH, CH)])
    pltpu.sync_copy(rowsv_a.at[pl.ds(0, SA - 2 * CH)],
                    acc.at[pl.ds(s * SA + 2 * CH, SA - 2 * CH)])
    plsc.subcore_barrier()

    coff = c

    def _outer(t, _):
        blk = s * (EW // CH) + t * GA
        pltpu.sync_copy(cols_g_ref.at[pl.ds(blk, GA)], cidx_blk)
        pltpu.sync_copy(rows_s_ref.at[pl.ds(blk, GA)], ridx_blk)

        def _gidx(q, dst):
            def _off(w, __):
                dst[pl.ds(w * 16, 16)] = (
                    cidx_blk[q, pl.ds(w * 16, 16)] * 2 + coff)
                return 0
            lax.fori_loop(0, CH // 16, _off, 0)

        gd = [None, None]
        sd = [None, None]
        _gidx(0, gidx[0])
        gd[0] = pltpu.async_copy(item_t_ref.at[gidx[0]], rowsv[0], gsem[0])
        for q in range(GA):
            p = q & 1
            pn = 1 - p
            if q + 1 < GA:
                if sd[pn] is not None:
                    sd[pn].wait()
                    sd[pn] = None
                _gidx(q + 1, gidx[pn])
                gd[pn] = pltpu.async_copy(
                    item_t_ref.at[gidx[pn]], rowsv[pn], gsem[pn])
            gd[p].wait()
            sd[p] = pltpu.async_copy(
                rowsv[p], acc.at[ridx_blk.at[q]], ssem[p], add=True)
        for b in range(2):
            if sd[b] is not None:
                sd[b].wait()
        return 0
    lax.fori_loop(0, TA, _outer, 0)
    plsc.subcore_barrier()

    def _wb(colo):
        for off, sz in ((0, CH), (CH, CH), (2 * CH, 275)):
            pltpu.sync_copy(acc.at[pl.ds(s * 1875 + off, sz)],
                            rowsv_a.at[pl.ds(0, sz)])
            pltpu.sync_copy(
                rowsv_a.at[pl.ds(0, sz)],
                hu_ref.at[pl.ds(s * 1875 + off, sz), pl.ds(colo, 32)])

    @pl.when(c == 0)
    def _wb0():
        _wb(0)

    @pl.when(c == 1)
    def _wb1():
        _wb(32)


_sc_spmm_u = functools.partial(
    pl.kernel,
    out_type=jax.ShapeDtypeStruct((NU, D), jnp.float32),
    mesh=_mesh,
    scratch_types=[
        pltpu.VMEM((GA, CH), jnp.int32),
        pltpu.VMEM((GA, CH), jnp.int32),
        pltpu.VMEM((CH,), jnp.int32),
        pltpu.VMEM((CH,), jnp.int32),
        pltpu.VMEM((CH, 32), jnp.float32),
        pltpu.VMEM((CH, 32), jnp.float32),
        pltpu.VMEM_SHARED((AU, 32), jnp.float32),
        pltpu.SemaphoreType.DMA,
        pltpu.SemaphoreType.DMA,
        pltpu.SemaphoreType.DMA,
        pltpu.SemaphoreType.DMA,
    ],
    compiler_params=_sc_params,
)(_sc_spmm_u_body)


def _sc_spmm_i_body(rows_g_ref, cols_s_ref, su_ref,
                    hi_ref,
                    ridx_blk, cidx_blk, gidx_a, gidx_b, gidx_c,
                    rowsv_a, rowsv_b, rowsv_c,
                    acc, semga, semgb, semgc, semsa, semsb, semsc):
    c = lax.axis_index("c")
    s = lax.axis_index("s")
    gidx = (gidx_a, gidx_b, gidx_c)
    rowsv = (rowsv_a, rowsv_b, rowsv_c)
    gsem = (semga, semgb, semgc)
    ssem = (semsa, semsb, semsc)
    for j in range(2):
        sl = c * 2 + j

        def _zrow(r, _):
            rowsv_a[r, pl.ds(0, 16)] = jnp.zeros((16,), jnp.float32)
            return 0
        lax.fori_loop(0, CH, _zrow, 0)

        for q in range(5):
            pltpu.sync_copy(rowsv_a, acc.at[pl.ds(s * 4500 + q * CH, CH)])
        pltpu.sync_copy(rowsv_a.at[pl.ds(0, 500)],
                        acc.at[pl.ds(s * 4500 + 5 * CH, 500)])
        plsc.subcore_barrier()
        soff = sl

        def _outer(t, _):
            blk = s * (EW // CH) + t * GB
            pltpu.sync_copy(rows_g_ref.at[pl.ds(blk, GB)], ridx_blk)
            pltpu.sync_copy(cols_s_ref.at[pl.ds(blk, GB)], cidx_blk)

            def _gidx(q, dst):
                def _off(w, __):
                    dst[pl.ds(w * 16, 16)] = (
                        ridx_blk[q, pl.ds(w * 16, 16)] * 4 + soff)
                    return 0
                lax.fori_loop(0, CH // 16, _off, 0)

            gd = [None, None, None]
            sd = [None, None, None]
            for q0 in range(2):
                _gidx(q0, gidx[q0])
                gd[q0] = pltpu.async_copy(
                    su_ref.at[gidx[q0]], rowsv[q0], gsem[q0])
            for q in range(GB):
                p = q % 3
                p2 = (q + 2) % 3
                if q + 2 < GB:
                    if sd[p2] is not None:
                        sd[p2].wait()
                        sd[p2] = None
                    _gidx(q + 2, gidx[p2])
                    gd[p2] = pltpu.async_copy(
                        su_ref.at[gidx[p2]], rowsv[p2], gsem[p2])
                gd[p].wait()
                sd[p] = pltpu.async_copy(
                    rowsv[p], acc.at[cidx_blk.at[q]], ssem[p], add=True)
            for b in range(3):
                if sd[b] is not None:
                    sd[b].wait()
            return 0
        lax.fori_loop(0, TB, _outer, 0)
        plsc.subcore_barrier()

        def _wb(colo):
            for off, sz in ((0, CH), (CH, CH), (2 * CH, CH), (3 * CH, CH),
                            (4 * CH, CH), (5 * CH, 375)):
                pltpu.sync_copy(acc.at[pl.ds(s * 4375 + off, sz)],
                                rowsv_a.at[pl.ds(0, sz)])
                pltpu.sync_copy(
                    rowsv_a.at[pl.ds(0, sz)],
                    hi_ref.at[pl.ds(s * 4375 + off, sz), pl.ds(colo, 16)])

        @pl.when(c == 0)
        def _wb0():
            _wb(16 * j)

        @pl.when(c == 1)
        def _wb1():
            _wb(16 * (2 + j))


_sc_spmm_i = functools.partial(
    pl.kernel,
    out_type=jax.ShapeDtypeStruct((NI, D), jnp.float32),
    mesh=_mesh,
    scratch_types=[
        pltpu.VMEM((GB, CH), jnp.int32),
        pltpu.VMEM((GB, CH), jnp.int32),
        pltpu.VMEM((CH,), jnp.int32),
        pltpu.VMEM((CH,), jnp.int32),
        pltpu.VMEM((CH,), jnp.int32),
        pltpu.VMEM((CH, 16), jnp.float32),
        pltpu.VMEM((CH, 16), jnp.float32),
        pltpu.VMEM((CH, 16), jnp.float32),
        pltpu.VMEM_SHARED((AI, 16), jnp.float32),
        pltpu.SemaphoreType.DMA,
        pltpu.SemaphoreType.DMA,
        pltpu.SemaphoreType.DMA,
        pltpu.SemaphoreType.DMA,
        pltpu.SemaphoreType.DMA,
        pltpu.SemaphoreType.DMA,
    ],
    compiler_params=_sc_params,
)(_sc_spmm_i_body)


_NBW = NB // (NC * NS)


def _sc_gather_body(hu_ref, hi_ref, deg_ref, bias_ref,
                    bu_ref, bp_ref, bn_ref,
                    hug_ref, hpg_ref, hng_ref,
                    bug_ref, bpg_ref, bng_ref, dg0_ref, dg1_ref,
                    idxv, idx2v, rowsv, degv, sem):
    c = lax.axis_index("c")
    s = lax.axis_index("s")
    n = _NBW
    base = (s * NC + c) * n

    def _shift(off):
        def _f(w, _):
            idx2v[pl.ds(w * 16, 16)] = idxv[pl.ds(w * 16, 16)] + off
            return 0
        lax.fori_loop(0, n // 16, _f, 0)

    pltpu.sync_copy(bu_ref.at[pl.ds(base, n)], idxv)
    pltpu.async_copy(hu_ref.at[idxv], rowsv, sem).wait()
    pltpu.sync_copy(rowsv, hug_ref.at[pl.ds(base, n)])
    pltpu.async_copy(bias_ref.at[idxv], rowsv, sem).wait()
    pltpu.sync_copy(rowsv, bug_ref.at[pl.ds(base, n)])
    pltpu.async_copy(deg_ref.at[idxv], degv, sem).wait()
    pltpu.sync_copy(degv, dg0_ref.at[pl.ds(base, n)])
    _shift(AU)
    pltpu.async_copy(deg_ref.at[idx2v], degv, sem).wait()
    pltpu.sync_copy(degv, dg1_ref.at[pl.ds(base, n)])

    for idx_hbm, hout, bout in ((bp_ref, hpg_ref, bpg_ref),
                                (bn_ref, hng_ref, bng_ref)):
        pltpu.sync_copy(idx_hbm.at[pl.ds(base, n)], idxv)
        pltpu.async_copy(hi_ref.at[idxv], rowsv, sem).wait()
        pltpu.sync_copy(rowsv, hout.at[pl.ds(base, n)])
        _shift(NU)
        pltpu.async_copy(bias_ref.at[idx2v], rowsv, sem).wait()
        pltpu.sync_copy(rowsv, bout.at[pl.ds(base, n)])


_sc_gather = functools.partial(
    pl.kernel,
    out_type=[jax.ShapeDtypeStruct((NB, D), jnp.float32)] * 6 + [
        jax.ShapeDtypeStruct((NB,), jnp.float32)] * 2,
    mesh=_mesh,
    scratch_types=[
        pltpu.VMEM((_NBW,), jnp.int32),
        pltpu.VMEM((_NBW,), jnp.int32),
        pltpu.VMEM((_NBW, D), jnp.float32),
        pltpu.VMEM((_NBW,), jnp.float32),
        pltpu.SemaphoreType.DMA,
    ],
    compiler_params=_sc_params,
)(_sc_gather_body)


def _t1_body(re_ref, mask_ref, ent_ref, re2_ref, mask2_ref, ent2_ref, w2_ref,
             rfi_ref, rfu_ref):
    def _rel(r, m, e):
        x = r[...]
        x = jnp.exp(x - jnp.max(x, axis=1, keepdims=True))
        sm = x / jnp.sum(x, axis=1, keepdims=True)
        return jnp.dot(sm * m[...], e[...], preferred_element_type=jnp.float32)
    rfi_ref[...] = jnp.dot(_rel(re_ref, mask_ref, ent_ref), w2_ref[...],
                           preferred_element_type=jnp.float32)
    rfu_ref[...] = jnp.dot(_rel(re2_ref, mask2_ref, ent2_ref), w2_ref[...],
                           preferred_element_type=jnp.float32)


def _t1(re, mask, ent, re2, mask2, ent2, w2):
    return pl.pallas_call(
        _t1_body,
        out_shape=[jax.ShapeDtypeStruct((16, D), jnp.float32),
                   jax.ShapeDtypeStruct((8, D), jnp.float32)],
    )(re, mask, ent, re2, mask2, ent2, w2)


_BLK = 2000


def _t2_body(sn_ref, ir_ref, w1_ref, rfi_ref, out_ref):
    out_ref[...] = (
        jnp.dot(sn_ref[...], w1_ref[...], preferred_element_type=jnp.float32)
        + jnp.dot(ir_ref[...], rfi_ref[...], preferred_element_type=jnp.float32))


def _t2(struct_node_emb, ir, w1, rfi):
    nb = NI // _BLK
    return pl.pallas_call(
        _t2_body,
        grid=(nb,),
        in_specs=[
            pl.BlockSpec((_BLK, D), lambda i: (NU // _BLK + i, 0)),
            pl.BlockSpec((_BLK, 16), lambda i: (i, 0)),
            pl.BlockSpec((D, D), lambda i: (0, 0)),
            pl.BlockSpec((16, D), lambda i: (0, 0)),
        ],
        out_specs=pl.BlockSpec((_BLK, D), lambda i: (i, 0)),
        out_shape=jax.ShapeDtypeStruct((NI, D), jnp.float32),
    )(struct_node_emb, ir, w1, rfi)


def _t3_body(sn_ref, ur_ref, w1_ref, rfu_ref, out_ref):
    out_ref[...] = (
        jnp.dot(sn_ref[...], w1_ref[...], preferred_element_type=jnp.float32)
        + jnp.dot(ur_ref[...], rfu_ref[...], preferred_element_type=jnp.float32))


def _t3(struct_node_emb, ur, w1, rfu):
    nb = NU // _BLK
    return pl.pallas_call(
        _t3_body,
        grid=(nb,),
        in_specs=[
            pl.BlockSpec((_BLK, D), lambda i: (i, 0)),
            pl.BlockSpec((_BLK, 8), lambda i: (i, 0)),
            pl.BlockSpec((D, D), lambda i: (0, 0)),
            pl.BlockSpec((8, D), lambda i: (0, 0)),
        ],
        out_specs=pl.BlockSpec((_BLK, D), lambda i: (i, 0)),
        out_shape=jax.ShapeDtypeStruct((NU, D), jnp.float32),
    )(struct_node_emb, ur, w1, rfu)


def _t4_body(uc_ref, d0_ref, d1_ref, out_ref):
    deg = d0_ref[...] + d1_ref[...]
    dinv = jnp.where(deg > 0, 1.0 / deg, 0.0)
    out_ref[...] = uc_ref[...] * dinv


def _t4(user_comb, deg0, deg1):
    nb = NU // _BLK
    return pl.pallas_call(
        _t4_body,
        grid=(nb,),
        in_specs=[
            pl.BlockSpec((_BLK, D), lambda i: (i, 0)),
            pl.BlockSpec((_BLK, 1), lambda i: (i, 0)),
            pl.BlockSpec((_BLK, 1), lambda i: (i, 0)),
        ],
        out_specs=pl.BlockSpec((_BLK, D), lambda i: (i, 0)),
        out_shape=jax.ShapeDtypeStruct((NU, D), jnp.float32),
    )(user_comb, deg0, deg1)


def _finalize(x):
    x = jnp.where(x > 0, x, 0.2 * x)
    n = jnp.sqrt(jnp.sum(x * x, axis=1, keepdims=True))
    return NLAYERS * x / jnp.maximum(n, 1e-12)


def _t5u_body(a_ref, d0_ref, d1_ref, bias_ref, out_ref):
    deg = d0_ref[...] + d1_ref[...]
    dinv = jnp.where(deg > 0, 1.0 / deg, 0.0)
    out_ref[...] = _finalize(a_ref[...] * dinv + bias_ref[...])


def _t5u(hu, deg0, deg1, bias):
    nb = NU // _BLK
    return pl.pallas_call(
        _t5u_body,
        grid=(nb,),
        in_specs=[
            pl.BlockSpec((_BLK, D), lambda i: (i, 0)),
            pl.BlockSpec((_BLK, 1), lambda i: (i, 0)),
            pl.BlockSpec((_BLK, 1), lambda i: (i, 0)),
            pl.BlockSpec((_BLK, D), lambda i: (i, 0)),
        ],
        out_specs=pl.BlockSpec((_BLK, D), lambda i: (i, 0)),
        out_shape=jax.ShapeDtypeStruct((NU, D), jnp.float32),
    )(hu, deg0, deg1, bias)


def _t5i_body(h_ref, bias_ref, out_ref):
    out_ref[...] = _finalize(h_ref[...] + bias_ref[...])


def _t5i(hi, bias):
    nb = NI // _BLK
    return pl.pallas_call(
        _t5i_body,
        grid=(nb,),
        in_specs=[
            pl.BlockSpec((_BLK, D), lambda i: (i, 0)),
            pl.BlockSpec((_BLK, D), lambda i: (NU // _BLK + i, 0)),
        ],
        out_specs=pl.BlockSpec((_BLK, D), lambda i: (i, 0)),
        out_shape=jax.ShapeDtypeStruct((NI, D), jnp.float32),
    )(hi, bias)


def _t6_body(hug_ref, hpg_ref, hng_ref, bug_ref, bpg_ref, bng_ref,
             d0_ref, d1_ref, out_ref):
    deg = d0_ref[...] + d1_ref[...]
    dinv = jnp.where(deg > 0, 1.0 / deg, 0.0)
    ug = _finalize(hug_ref[...] * dinv + bug_ref[...])
    pg = _finalize(hpg_ref[...] + bpg_ref[...])
    ng = _finalize(hng_ref[...] + bng_ref[...])
    ps = jnp.sum(ug * pg, axis=1, keepdims=True)
    ns = jnp.sum(ug * ng, axis=1, keepdims=True)
    x = ps - ns
    ls = jnp.minimum(x, 0.0) - jnp.log(1.0 + jnp.exp(-jnp.abs(x)))
    mf = -jnp.sum(ls) / NB
    reg = (jnp.sum(ug * ug) + jnp.sum(pg * pg) + jnp.sum(ng * ng)) * 0.5
    out_ref[...] = jnp.reshape(mf + DECAY * reg / NB, (1, 1))


def _t6(hug, hpg, hng, bug, bpg, bng, dg0, dg1):
    return pl.pallas_call(
        _t6_body,
        out_shape=jax.ShapeDtypeStruct((1, 1), jnp.float32),
    )(hug, hpg, hng, bug, bpg, bng,
      dg0.reshape(NB, 1), dg1.reshape(NB, 1))


def kernel(struct_node_emb, train_weight, train_weight_2, bias, re, entity_emb,
           ir, re_2, entity_emb_2, ur, mask, mask_2, rows, cols,
           batch_users, batch_pos, batch_neg):
    rows = rows.astype(jnp.int32)
    cols = cols.astype(jnp.int32)
    pad = jnp.arange(NPAD, dtype=jnp.int32)
    blk2 = (EPAD // CH, CH)
    rows_s = jnp.concatenate([rows, NU + pad % NDUMU]).reshape(blk2)
    rows_g = jnp.concatenate([rows, pad % 16]).reshape(blk2)
    cols_s = jnp.concatenate([cols, NI + pad % NDUMI]).reshape(blk2)
    cols_g = jnp.concatenate([cols, pad % 16]).reshape(blk2)

    rfi, rfu = _t1(re, mask, entity_emb, re_2, mask_2, entity_emb_2,
                   train_weight_2)
    item_t = _t2(struct_node_emb, ir, train_weight, rfi).reshape(2 * NI, 32)
    user_comb = _t3(struct_node_emb, ur, train_weight, rfu)

    deg_flat = _sc_deg(rows_s)
    hu64 = _sc_spmm_u(rows_s, cols_g, item_t)

    deg0 = deg_flat[:NU].reshape(NU, 1)
    deg1 = deg_flat[AU:AU + NU].reshape(NU, 1)
    su = _t4(user_comb, deg0, deg1).reshape(4 * NU, 16)

    hi64 = _sc_spmm_i(rows_g, cols_s, su)

    u = _t5u(hu64, deg0, deg1, bias)
    iv = _t5i(hi64, bias)

    hug, hpg, hng, bug, bpg, bng, dg0, dg1 = _sc_gather(
        hu64, hi64, deg_flat, bias,
        batch_users.astype(jnp.int32), batch_pos.astype(jnp.int32),
        batch_neg.astype(jnp.int32))
    loss = _t6(hug, hpg, hng, bug, bpg, bng, dg0, dg1).reshape(())
    return (loss, u, iv)


# speedup vs baseline: 1.0458x; 1.0458x over previous
import functools

import jax
import jax.numpy as jnp
from jax import lax
from jax.experimental import pallas as pl
from jax.experimental.pallas import tpu as pltpu
from jax.experimental.pallas import tpu_sc as plsc

NU = 30000
NI = 70000
D = 64
NNZ = 1000000
NB = 4096
NLAYERS = 3.0
DECAY = 1e-4

NC = 2
NS = 16
CH = 800
EPAD = 1024000
NPAD = EPAD - NNZ
NDUMU = 704
NDUMI = 1600
AU = NU + 720
AI = NI + 2000
SA = AU // NS
EW = EPAD // NS
NCH = EW // CH
GA = 4
GB = 10
TA = NCH // GA
TB = NCH // GB

_mesh = plsc.VectorSubcoreMesh(core_axis_name="c", subcore_axis_name="s",
                               num_cores=NC, num_subcores=NS)
_sc_params = pltpu.CompilerParams(use_tc_tiling_on_sc=False)


def _sc_deg_body(rows_s_ref, deg_ref, didx_a, didx_b, onesv, zdeg, dacc, semd):
    c = lax.axis_index("c")
    s = lax.axis_index("s")
    didx = (didx_a, didx_b)

    def _init(t, _):
        onesv[pl.ds(t * 16, 16)] = jnp.full((16,), 1.0, jnp.float32)
        return 0
    lax.fori_loop(0, CH // 16, _init, 0)

    def _zdeg(t, _):
        zdeg[pl.ds(t * 16, 16)] = jnp.zeros((16,), jnp.float32)
        return 0
    lax.fori_loop(0, SA // 16, _zdeg, 0)

    pltpu.sync_copy(zdeg, dacc.at[pl.ds(s * SA, SA)])
    plsc.subcore_barrier()

    row0 = (c * NS + s) * (EPAD // (NC * NS) // CH)
    nrows = EPAD // (NC * NS) // CH
    sd = [None, None]
    for q in range(nrows):
        p = q & 1
        if sd[p] is not None:
            sd[p].wait()
        pltpu.sync_copy(rows_s_ref.at[pl.ds(row0 + q, 1)], didx[p])
        sd[p] = pltpu.async_copy(onesv, dacc.at[didx[p].at[0]], semd,
                                 add=True)
    for b in range(2):
        if sd[b] is not None:
            sd[b].wait()
    plsc.subcore_barrier()
    pltpu.sync_copy(dacc.at[pl.ds(s * SA, SA)], zdeg)
    pltpu.sync_copy(zdeg, deg_ref.at[pl.ds(c * AU + s * SA, SA)])


_sc_deg = functools.partial(
    pl.kernel,
    out_type=jax.ShapeDtypeStruct((NC * AU,), jnp.float32),
    mesh=_mesh,
    scratch_types=[
        pltpu.VMEM((1, CH), jnp.int32),
        pltpu.VMEM((1, CH), jnp.int32),
        pltpu.VMEM((CH,), jnp.float32),
        pltpu.VMEM((SA,), jnp.float32),
        pltpu.VMEM_SHARED((AU,), jnp.float32),
        pltpu.SemaphoreType.DMA,
    ],
    compiler_params=_sc_params,
)(_sc_deg_body)


def _sc_spmm_u_body(rows_s_ref, cols_g_ref, item_t_ref,
                    hu_ref,
                    cidx_blk, ridx_blk, gidx_a, gidx_b,
                    rowsv_a, rowsv_b,
                    acc, semga, semgb, semsa, semsb):
    c = lax.axis_index("c")
    s = lax.axis_index("s")
    gidx = (gidx_a, gidx_b)
    rowsv = (rowsv_a, rowsv_b)
    gsem = (semga, semgb)
    ssem = (semsa, semsb)

    def _zrow(r, _):
        rowsv_a[r, pl.ds(0, 16)] = jnp.zeros((16,), jnp.float32)
        rowsv_a[r, pl.ds(16, 16)] = jnp.zeros((16,), jnp.float32)
        return 0
    lax.fori_loop(0, CH, _zrow, 0)

    pltpu.sync_copy(rowsv_a, acc.at[pl.ds(s * SA, CH)])
    pltpu.sync_copy(rowsv_a, acc.at[pl.ds(s * SA + CH, CH)])
    pltpu.sync_copy(rowsv_a.at[pl.ds(0, SA - 2 * CH)],
                    acc.at[pl.ds(s * SA + 2 * CH, SA - 2 * CH)])
    plsc.subcore_barrier()

    coff = c

    def _outer(t, _):
        blk = s * (EW // CH) + t * GA
        pltpu.sync_copy(cols_g_ref.at[pl.ds(blk, GA)], cidx_blk)
        pltpu.sync_copy(rows_s_ref.at[pl.ds(blk, GA)], ridx_blk)

        def _gidx(q, dst):
            def _off(w, __):
                dst[pl.ds(w * 16, 16)] = (
                    cidx_blk[q, pl.ds(w * 16, 16)] * 2 + coff)
                return 0
            lax.fori_loop(0, CH // 16, _off, 0)

        gd = [None, None]
        sd = [None, None]
        _gidx(0, gidx[0])
        gd[0] = pltpu.async_copy(item_t_ref.at[gidx[0]], rowsv[0], gsem[0])
        for q in range(GA):
            p = q & 1
            pn = 1 - p
            if q + 1 < GA:
                if sd[pn] is not None:
                    sd[pn].wait()
                    sd[pn] = None
                _gidx(q + 1, gidx[pn])
                gd[pn] = pltpu.async_copy(
                    item_t_ref.at[gidx[pn]], rowsv[pn], gsem[pn])
            gd[p].wait()
            sd[p] = pltpu.async_copy(
                rowsv[p], acc.at[ridx_blk.at[q]], ssem[p], add=True)
        for b in range(2):
            if sd[b] is not None:
                sd[b].wait()
        return 0
    lax.fori_loop(0, TA, _outer, 0)
    plsc.subcore_barrier()

    def _wb(colo):
        for off, sz in ((0, CH), (CH, CH), (2 * CH, 275)):
            pltpu.sync_copy(acc.at[pl.ds(s * 1875 + off, sz)],
                            rowsv_a.at[pl.ds(0, sz)])
            pltpu.sync_copy(
                rowsv_a.at[pl.ds(0, sz)],
                hu_ref.at[pl.ds(s * 1875 + off, sz), pl.ds(colo, 32)])

    @pl.when(c == 0)
    def _wb0():
        _wb(0)

    @pl.when(c == 1)
    def _wb1():
        _wb(32)


_sc_spmm_u = functools.partial(
    pl.kernel,
    out_type=jax.ShapeDtypeStruct((NU, D), jnp.float32),
    mesh=_mesh,
    scratch_types=[
        pltpu.VMEM((GA, CH), jnp.int32),
        pltpu.VMEM((GA, CH), jnp.int32),
        pltpu.VMEM((CH,), jnp.int32),
        pltpu.VMEM((CH,), jnp.int32),
        pltpu.VMEM((CH, 32), jnp.float32),
        pltpu.VMEM((CH, 32), jnp.float32),
        pltpu.VMEM_SHARED((AU, 32), jnp.float32),
        pltpu.SemaphoreType.DMA,
        pltpu.SemaphoreType.DMA,
        pltpu.SemaphoreType.DMA,
        pltpu.SemaphoreType.DMA,
    ],
    compiler_params=_sc_params,
)(_sc_spmm_u_body)


def _sc_spmm_i_body(rows_g_ref, cols_s_ref, su_ref,
                    hi_ref,
                    ridx_blk, cidx_blk, gidx_a, gidx_b, gidx_c,
                    rowsv_a, rowsv_b, rowsv_c,
                    acc, semga, semgb, semgc, semsa, semsb, semsc):
    c = lax.axis_index("c")
    s = lax.axis_index("s")
    gidx = (gidx_a, gidx_b, gidx_c)
    rowsv = (rowsv_a, rowsv_b, rowsv_c)
    gsem = (semga, semgb, semgc)
    ssem = (semsa, semsb, semsc)
    for j in range(2):
        sl = c * 2 + j

        def _zrow(r, _):
            rowsv_a[r, pl.ds(0, 16)] = jnp.zeros((16,), jnp.float32)
            return 0
        lax.fori_loop(0, CH, _zrow, 0)

        for q in range(5):
            pltpu.sync_copy(rowsv_a, acc.at[pl.ds(s * 4500 + q * CH, CH)])
        pltpu.sync_copy(rowsv_a.at[pl.ds(0, 500)],
                        acc.at[pl.ds(s * 4500 + 5 * CH, 500)])
        plsc.subcore_barrier()
        soff = sl

        def _outer(t, _):
            blk = s * (EW // CH) + t * GB
            pltpu.sync_copy(rows_g_ref.at[pl.ds(blk, GB)], ridx_blk)
            pltpu.sync_copy(cols_s_ref.at[pl.ds(blk, GB)], cidx_blk)

            def _gidx(q, dst):
                def _off(w, __):
                    dst[pl.ds(w * 16, 16)] = (
                        ridx_blk[q, pl.ds(w * 16, 16)] * 4 + soff)
                    return 0
                lax.fori_loop(0, CH // 16, _off, 0)

            gd = [None, None, None]
            sd = [None, None, None]
            for q0 in range(2):
                _gidx(q0, gidx[q0])
                gd[q0] = pltpu.async_copy(
                    su_ref.at[gidx[q0]], rowsv[q0], gsem[q0])
            for q in range(GB):
                p = q % 3
                p2 = (q + 2) % 3
                if q + 2 < GB:
                    if sd[p2] is not None:
                        sd[p2].wait()
                        sd[p2] = None
                    _gidx(q + 2, gidx[p2])
                    gd[p2] = pltpu.async_copy(
                        su_ref.at[gidx[p2]], rowsv[p2], gsem[p2])
                gd[p].wait()
                sd[p] = pltpu.async_copy(
                    rowsv[p], acc.at[cidx_blk.at[q]], ssem[p], add=True)
            for b in range(3):
                if sd[b] is not None:
                    sd[b].wait()
            return 0
        lax.fori_loop(0, TB, _outer, 0)
        plsc.subcore_barrier()

        def _wb(colo):
            for off, sz in ((0, CH), (CH, CH), (2 * CH, CH), (3 * CH, CH),
                            (4 * CH, CH), (5 * CH, 375)):
                pltpu.sync_copy(acc.at[pl.ds(s * 4375 + off, sz)],
                                rowsv_a.at[pl.ds(0, sz)])
                pltpu.sync_copy(
                    rowsv_a.at[pl.ds(0, sz)],
                    hi_ref.at[pl.ds(s * 4375 + off, sz), pl.ds(colo, 16)])

        @pl.when(c == 0)
        def _wb0():
            _wb(16 * j)

        @pl.when(c == 1)
        def _wb1():
            _wb(16 * (2 + j))


_sc_spmm_i = functools.partial(
    pl.kernel,
    out_type=jax.ShapeDtypeStruct((NI, D), jnp.float32),
    mesh=_mesh,
    scratch_types=[
        pltpu.VMEM((GB, CH), jnp.int32),
        pltpu.VMEM((GB, CH), jnp.int32),
        pltpu.VMEM((CH,), jnp.int32),
        pltpu.VMEM((CH,), jnp.int32),
        pltpu.VMEM((CH,), jnp.int32),
        pltpu.VMEM((CH, 16), jnp.float32),
        pltpu.VMEM((CH, 16), jnp.float32),
        pltpu.VMEM((CH, 16), jnp.float32),
        pltpu.VMEM_SHARED((AI, 16), jnp.float32),
        pltpu.SemaphoreType.DMA,
        pltpu.SemaphoreType.DMA,
        pltpu.SemaphoreType.DMA,
        pltpu.SemaphoreType.DMA,
        pltpu.SemaphoreType.DMA,
        pltpu.SemaphoreType.DMA,
    ],
    compiler_params=_sc_params,
)(_sc_spmm_i_body)


def _sc_gather_body(u_ref, i_ref, bu_ref, bp_ref, bn_ref,
                    ug_ref, pg_ref, ng_ref,
                    idxv, rowsv, sem):
    c = lax.axis_index("c")
    s = lax.axis_index("s")
    n = NB // (NC * NS)
    base = (s * NC + c) * n
    for src, idx_hbm, out in ((u_ref, bu_ref, ug_ref),
                              (i_ref, bp_ref, pg_ref),
                              (i_ref, bn_ref, ng_ref)):
        pltpu.sync_copy(idx_hbm.at[pl.ds(base, n)], idxv)
        pltpu.async_copy(src.at[idxv], rowsv, sem).wait()
        pltpu.sync_copy(rowsv, out.at[pl.ds(base, n)])


_sc_gather = functools.partial(
    pl.kernel,
    out_type=[jax.ShapeDtypeStruct((NB, D), jnp.float32)] * 3,
    mesh=_mesh,
    scratch_types=[
        pltpu.VMEM((NB // (NC * NS),), jnp.int32),
        pltpu.VMEM((NB // (NC * NS), D), jnp.float32),
        pltpu.SemaphoreType.DMA,
    ],
    compiler_params=_sc_params,
)(_sc_gather_body)


def _t1_body(re_ref, mask_ref, ent_ref, re2_ref, mask2_ref, ent2_ref, w2_ref,
             rfi_ref, rfu_ref):
    def _rel(r, m, e):
        x = r[...]
        x = jnp.exp(x - jnp.max(x, axis=1, keepdims=True))
        sm = x / jnp.sum(x, axis=1, keepdims=True)
        return jnp.dot(sm * m[...], e[...], preferred_element_type=jnp.float32)
    rfi_ref[...] = jnp.dot(_rel(re_ref, mask_ref, ent_ref), w2_ref[...],
                           preferred_element_type=jnp.float32)
    rfu_ref[...] = jnp.dot(_rel(re2_ref, mask2_ref, ent2_ref), w2_ref[...],
                           preferred_element_type=jnp.float32)


def _t1(re, mask, ent, re2, mask2, ent2, w2):
    return pl.pallas_call(
        _t1_body,
        out_shape=[jax.ShapeDtypeStruct((16, D), jnp.float32),
                   jax.ShapeDtypeStruct((8, D), jnp.float32)],
    )(re, mask, ent, re2, mask2, ent2, w2)


_BLK = 2000


def _t2_body(sn_ref, ir_ref, w1_ref, rfi_ref, out_ref):
    out_ref[...] = (
        jnp.dot(sn_ref[...], w1_ref[...], preferred_element_type=jnp.float32)
        + jnp.dot(ir_ref[...], rfi_ref[...], preferred_element_type=jnp.float32))


def _t2(struct_node_emb, ir, w1, rfi):
    nb = NI // _BLK
    return pl.pallas_call(
        _t2_body,
        grid=(nb,),
        in_specs=[
            pl.BlockSpec((_BLK, D), lambda i: (NU // _BLK + i, 0)),
            pl.BlockSpec((_BLK, 16), lambda i: (i, 0)),
            pl.BlockSpec((D, D), lambda i: (0, 0)),
            pl.BlockSpec((16, D), lambda i: (0, 0)),
        ],
        out_specs=pl.BlockSpec((_BLK, D), lambda i: (i, 0)),
        out_shape=jax.ShapeDtypeStruct((NI, D), jnp.float32),
    )(struct_node_emb, ir, w1, rfi)


def _t3_body(sn_ref, ur_ref, w1_ref, rfu_ref, out_ref):
    out_ref[...] = (
        jnp.dot(sn_ref[...], w1_ref[...], preferred_element_type=jnp.float32)
        + jnp.dot(ur_ref[...], rfu_ref[...], preferred_element_type=jnp.float32))


def _t3(struct_node_emb, ur, w1, rfu):
    nb = NU // _BLK
    return pl.pallas_call(
        _t3_body,
        grid=(nb,),
        in_specs=[
            pl.BlockSpec((_BLK, D), lambda i: (i, 0)),
            pl.BlockSpec((_BLK, 8), lambda i: (i, 0)),
            pl.BlockSpec((D, D), lambda i: (0, 0)),
            pl.BlockSpec((8, D), lambda i: (0, 0)),
        ],
        out_specs=pl.BlockSpec((_BLK, D), lambda i: (i, 0)),
        out_shape=jax.ShapeDtypeStruct((NU, D), jnp.float32),
    )(struct_node_emb, ur, w1, rfu)


def _t4_body(uc_ref, d0_ref, d1_ref, out_ref):
    deg = d0_ref[...] + d1_ref[...]
    dinv = jnp.where(deg > 0, 1.0 / deg, 0.0)
    out_ref[...] = uc_ref[...] * dinv


def _t4(user_comb, deg0, deg1):
    nb = NU // _BLK
    return pl.pallas_call(
        _t4_body,
        grid=(nb,),
        in_specs=[
            pl.BlockSpec((_BLK, D), lambda i: (i, 0)),
            pl.BlockSpec((_BLK, 1), lambda i: (i, 0)),
            pl.BlockSpec((_BLK, 1), lambda i: (i, 0)),
        ],
        out_specs=pl.BlockSpec((_BLK, D), lambda i: (i, 0)),
        out_shape=jax.ShapeDtypeStruct((NU, D), jnp.float32),
    )(user_comb, deg0, deg1)


def _finalize(x):
    x = jnp.where(x > 0, x, 0.2 * x)
    n = jnp.sqrt(jnp.sum(x * x, axis=1, keepdims=True))
    return NLAYERS * x / jnp.maximum(n, 1e-12)


def _t5u_body(a_ref, d0_ref, d1_ref, bias_ref, out_ref):
    deg = d0_ref[...] + d1_ref[...]
    dinv = jnp.where(deg > 0, 1.0 / deg, 0.0)
    out_ref[...] = _finalize(a_ref[...] * dinv + bias_ref[...])


def _t5u(hu, deg0, deg1, bias):
    nb = NU // _BLK
    return pl.pallas_call(
        _t5u_body,
        grid=(nb,),
        in_specs=[
            pl.BlockSpec((_BLK, D), lambda i: (i, 0)),
            pl.BlockSpec((_BLK, 1), lambda i: (i, 0)),
            pl.BlockSpec((_BLK, 1), lambda i: (i, 0)),
            pl.BlockSpec((_BLK, D), lambda i: (i, 0)),
        ],
        out_specs=pl.BlockSpec((_BLK, D), lambda i: (i, 0)),
        out_shape=jax.ShapeDtypeStruct((NU, D), jnp.float32),
    )(hu, deg0, deg1, bias)


def _t5i_body(h_ref, bias_ref, out_ref):
    out_ref[...] = _finalize(h_ref[...] + bias_ref[...])


def _t5i(hi, bias):
    nb = NI // _BLK
    return pl.pallas_call(
        _t5i_body,
        grid=(nb,),
        in_specs=[
            pl.BlockSpec((_BLK, D), lambda i: (i, 0)),
            pl.BlockSpec((_BLK, D), lambda i: (NU // _BLK + i, 0)),
        ],
        out_specs=pl.BlockSpec((_BLK, D), lambda i: (i, 0)),
        out_shape=jax.ShapeDtypeStruct((NI, D), jnp.float32),
    )(hi, bias)


def _t6_body(ug_ref, pg_ref, ng_ref, out_ref):
    ug, pg, ng = ug_ref[...], pg_ref[...], ng_ref[...]
    ps = jnp.sum(ug * pg, axis=1, keepdims=True)
    ns = jnp.sum(ug * ng, axis=1, keepdims=True)
    x = ps - ns
    ls = jnp.minimum(x, 0.0) - jnp.log(1.0 + jnp.exp(-jnp.abs(x)))
    mf = -jnp.sum(ls) / NB
    reg = (jnp.sum(ug * ug) + jnp.sum(pg * pg) + jnp.sum(ng * ng)) * 0.5
    out_ref[...] = jnp.reshape(mf + DECAY * reg / NB, (1, 1))


def _t6(ug, pg, ng):
    return pl.pallas_call(
        _t6_body,
        out_shape=jax.ShapeDtypeStruct((1, 1), jnp.float32),
    )(ug, pg, ng)


def kernel(struct_node_emb, train_weight, train_weight_2, bias, re, entity_emb,
           ir, re_2, entity_emb_2, ur, mask, mask_2, rows, cols,
           batch_users, batch_pos, batch_neg):
    rows = rows.astype(jnp.int32)
    cols = cols.astype(jnp.int32)
    pad = jnp.arange(NPAD, dtype=jnp.int32)
    blk2 = (EPAD // CH, CH)
    rows_s = jnp.concatenate([rows, NU + pad % NDUMU]).reshape(blk2)
    rows_g = jnp.concatenate([rows, pad % 16]).reshape(blk2)
    cols_s = jnp.concatenate([cols, NI + pad % NDUMI]).reshape(blk2)
    cols_g = jnp.concatenate([cols, pad % 16]).reshape(blk2)

    rfi, rfu = _t1(re, mask, entity_emb, re_2, mask_2, entity_emb_2,
                   train_weight_2)
    item_t = _t2(struct_node_emb, ir, train_weight, rfi).reshape(2 * NI, 32)
    user_comb = _t3(struct_node_emb, ur, train_weight, rfu)

    deg_flat = _sc_deg(rows_s)
    hu64 = _sc_spmm_u(rows_s, cols_g, item_t)

    deg0 = deg_flat[:NU].reshape(NU, 1)
    deg1 = deg_flat[AU:AU + NU].reshape(NU, 1)
    su = _t4(user_comb, deg0, deg1).reshape(4 * NU, 16)

    hi64 = _sc_spmm_i(rows_g, cols_s, su)

    u = _t5u(hu64, deg0, deg1, bias)
    iv = _t5i(hi64, bias)

    ug, pg, ng = _sc_gather(u, iv, batch_users.astype(jnp.int32),
                            batch_pos.astype(jnp.int32),
                            batch_neg.astype(jnp.int32))
    loss = _t6(ug, pg, ng).reshape(())
    return (loss, u, iv)
